# baseline (speedup 1.0000x reference)
import functools

import jax
import jax.numpy as jnp
from jax import lax
from jax.experimental import pallas as pl
from jax.experimental.pallas import tpu as pltpu
from jax.experimental.pallas import tpu_sc as plsc

D = 64
DP = 128
NT = 200
NB = 4096
V = 1000000
VMAIN = 999936
NPAIR = V // 2
QTAIL = VMAIN // 2

_info = plsc.get_sparse_core_info()
NC, NS = _info.num_cores, _info.num_subcores
NW = NC * NS

W = 384
WP = W // 2
NCH = VMAIN // W
ITER_A = 82

BLK = 128
NUNIT = NT

_mesh = plsc.VectorSubcoreMesh(core_axis_name="c", subcore_axis_name="s")
_params = pltpu.CompilerParams(use_tc_tiling_on_sc=True,
                               needs_layout_passes=False)


def _iota16():
    return lax.iota(jnp.int32, 16)


@functools.partial(
    pl.kernel,
    mesh=_mesh,
    out_type=jax.ShapeDtypeStruct((NPAIR, DP), jnp.float32),
    scratch_types=[
        pltpu.VMEM((3, D, 128), jnp.float32),
        pltpu.VMEM((3, D, 128), jnp.float32),
        pltpu.VMEM((WP, DP), jnp.float32),
        pltpu.VMEM((WP, DP), jnp.float32),
        pltpu.SemaphoreType.DMA,
        pltpu.SemaphoreType.DMA,
        pltpu.SemaphoreType.DMA,
        pltpu.SemaphoreType.DMA,
    ],
    compiler_params=_params,
)
def _pack_kernel(weT, tailp, tbl2, buf_a, buf_b, tb_a, tb_b,
                 lsem_a, lsem_b, ssem_a, ssem_b):
    wid = lax.axis_index("s") * NC + lax.axis_index("c")
    iota = _iota16()

    def load_start(c, buf, sem):
        for j in range(3):
            pltpu.async_copy(weT.at[:, pl.ds(c * W + j * 128, 128)],
                             buf.at[j], sem)

    def load_wait(buf, sem):
        for j in range(3):
            pltpu.make_async_copy(weT.at[:, pl.ds(0, 128)], buf.at[j],
                                  sem).wait()

    def store_start(c, tb, sem):
        pltpu.async_copy(tb, tbl2.at[pl.ds(c * WP, WP)], sem)

    def store_wait(tb, sem):
        pltpu.make_async_copy(tb, tbl2.at[pl.ds(0, WP)], sem).wait()

    def transpose(buf, tb):
        for j in range(3):

            def pbody(pg, _, j=j):
                for p2 in range(8):
                    p = pg * 8 + p2
                    for lg in range(4):
                        rvec = 16 * lg + iota
                        clo = jnp.full((16,), 2 * p, jnp.int32)
                        lo = plsc.load_gather(buf.at[j], [rvec, clo])
                        tb[64 * j + p, pl.ds(16 * lg, 16)] = lo
                        hi = plsc.load_gather(buf.at[j], [rvec, clo + 1])
                        tb[64 * j + p, pl.ds(64 + 16 * lg, 16)] = hi
                return _

            lax.fori_loop(0, 8, pbody, 0)

    def valid(k):
        return k * NW + wid < NCH

    load_start(wid, buf_a, lsem_a)

    def body(i, carry):
        k_a, k_b = 2 * i, 2 * i + 1
        c_a = k_a * NW + wid
        c_b = k_b * NW + wid

        @pl.when(valid(k_b))
        def _():
            load_start(c_b, buf_b, lsem_b)

        load_wait(buf_a, lsem_a)

        @pl.when(i > 0)
        def _():
            store_wait(tb_a, ssem_a)

        transpose(buf_a, tb_a)
        store_start(c_a, tb_a, ssem_a)

        @pl.when(k_a + 2 < ITER_A)
        def _():
            load_start(c_a + 2 * NW, buf_a, lsem_a)

        @pl.when(valid(k_b))
        def _():
            @pl.when(i > 0)
            def _():
                store_wait(tb_b, ssem_b)

            load_wait(buf_b, lsem_b)
            transpose(buf_b, tb_b)
            store_start(c_b, tb_b, ssem_b)

        return carry

    lax.fori_loop(0, ITER_A // 2, body, 0)

    store_wait(tb_a, ssem_a)

    @pl.when(valid(ITER_A - 1))
    def _():
        store_wait(tb_b, ssem_b)

    @pl.when(wid == 0)
    def _():
        pltpu.sync_copy(tailp, buf_a.at[0])
        for lg in range(4):
            rvec = 16 * lg + iota

            def tbody(p, _, lg=lg, rvec=rvec):
                pvec = jnp.full((16,), 2 * p, jnp.int32)
                lo = plsc.load_gather(buf_a.at[0], [pvec, rvec])
                tb_a[p, pl.ds(16 * lg, 16)] = lo
                hi = plsc.load_gather(buf_a.at[0], [pvec + 1, rvec])
                tb_a[p, pl.ds(64 + 16 * lg, 16)] = hi
                return _

            lax.fori_loop(0, 32, tbody, 0)
        pltpu.sync_copy(tb_a.at[pl.ds(0, 32)], tbl2.at[pl.ds(QTAIL, 32)])


@functools.partial(
    pl.kernel,
    mesh=_mesh,
    out_type=jax.ShapeDtypeStruct((NT, D, NB), jnp.float32),
    scratch_types=[
        pltpu.VMEM((BLK,), jnp.int32),
        pltpu.VMEM((BLK,), jnp.int32),
        pltpu.VMEM((BLK,), jnp.int32),
        pltpu.VMEM((BLK,), jnp.int32),
        pltpu.VMEM((BLK, DP), jnp.float32),
        pltpu.VMEM((BLK, DP), jnp.float32),
        pltpu.VMEM((D, BLK), jnp.float32),
        pltpu.VMEM((D, BLK), jnp.float32),
        pltpu.SemaphoreType.DMA,
        pltpu.SemaphoreType.DMA,
        pltpu.SemaphoreType.DMA,
        pltpu.SemaphoreType.DMA,
    ],
    compiler_params=_params,
)
def _gather_kernel(idxT, tbl2, outT, iv_a, iv_b, qv_a, qv_b,
                   buf_a, buf_b, tb_a, tb_b,
                   gsem_a, gsem_b, ssem_a, ssem_b):
    wid = lax.axis_index("s") * NC + lax.axis_index("c")
    b0 = wid * BLK
    iota = _iota16()

    def fetch_start(u, iv, qv, buf, sem):
        pltpu.sync_copy(idxT.at[u, pl.ds(b0, BLK)], iv)
        for g in range(8):
            qv[pl.ds(16 * g, 16)] = iv[pl.ds(16 * g, 16)] >> 1
        pltpu.async_copy(tbl2.at[qv], buf, sem)

    def fetch_wait(buf, sem):
        pltpu.make_async_copy(tbl2.at[qv_a], buf, sem).wait()

    def store_start(u, tb, sem):
        pltpu.async_copy(tb, outT.at[u, :, pl.ds(b0, BLK)], sem)

    def store_wait(tb, sem):
        pltpu.make_async_copy(tb, outT.at[0, :, pl.ds(b0, BLK)], sem).wait()

    def transpose(buf, iv, tb):
        for jg in range(8):
            jvec = 16 * jg + iota
            parv = (iv[pl.ds(16 * jg, 16)] & 1) * 64
            for d in range(D):
                v = plsc.load_gather(buf, [jvec, parv + d])
                tb[d, pl.ds(16 * jg, 16)] = v

    fetch_start(0, iv_a, qv_a, buf_a, gsem_a)

    def body(i, carry):
        u_a, u_b = 2 * i, 2 * i + 1
        fetch_start(u_b, iv_b, qv_b, buf_b, gsem_b)
        fetch_wait(buf_a, gsem_a)

        @pl.when(i > 0)
        def _():
            store_wait(tb_a, ssem_a)

        transpose(buf_a, iv_a, tb_a)
        store_start(u_a, tb_a, ssem_a)

        @pl.when(u_a + 2 < NUNIT)
        def _():
            fetch_start(u_a + 2, iv_a, qv_a, buf_a, gsem_a)

        fetch_wait(buf_b, gsem_b)

        @pl.when(i > 0)
        def _():
            store_wait(tb_b, ssem_b)

        transpose(buf_b, iv_b, tb_b)
        store_start(u_b, tb_b, ssem_b)
        return carry

    lax.fori_loop(0, NUNIT // 2, body, 0)

    store_wait(tb_a, ssem_a)
    store_wait(tb_b, ssem_b)


def kernel(input, word_embedding):
    weT = word_embedding.T
    tailp = jnp.pad(word_embedding[VMAIN:], ((0, 0), (0, DP - D)))
    tbl2 = _pack_kernel(weT, tailp)
    idxT = input.astype(jnp.int32).T
    outT = _gather_kernel(idxT, tbl2)
    return outT.transpose(2, 0, 1)


# speedup vs baseline: 1.8192x; 1.8192x over previous
import functools

import jax
import jax.numpy as jnp
from jax import lax
from jax.experimental import pallas as pl
from jax.experimental.pallas import tpu as pltpu
from jax.experimental.pallas import tpu_sc as plsc

D = 64
DP = 128
NT = 200
NB = 4096
V = 1000000
VMAIN = 999936
NPAIR = V // 2
QTAIL = VMAIN // 2

_info = plsc.get_sparse_core_info()
NC, NS = _info.num_cores, _info.num_subcores
NW = NC * NS

W = 384
WP = W // 2
NCH = VMAIN // W
ITER_A = 82

BLK = 128
NUNIT = NT

_mesh = plsc.VectorSubcoreMesh(core_axis_name="c", subcore_axis_name="s")
_params = pltpu.CompilerParams(use_tc_tiling_on_sc=True,
                               needs_layout_passes=False)


def _iota16():
    return lax.iota(jnp.int32, 16)


@functools.partial(
    pl.kernel,
    mesh=_mesh,
    out_type=jax.ShapeDtypeStruct((NPAIR, DP), jnp.float32),
    scratch_types=[
        pltpu.VMEM((3, D, 128), jnp.float32),
        pltpu.VMEM((3, D, 128), jnp.float32),
        pltpu.VMEM((WP, DP), jnp.float32),
        pltpu.VMEM((WP, DP), jnp.float32),
        pltpu.SemaphoreType.DMA,
        pltpu.SemaphoreType.DMA,
        pltpu.SemaphoreType.DMA,
        pltpu.SemaphoreType.DMA,
    ],
    compiler_params=_params,
)
def _pack_kernel(weT, tailp, tbl2, buf_a, buf_b, tb_a, tb_b,
                 lsem_a, lsem_b, ssem_a, ssem_b):
    wid = lax.axis_index("s") * NC + lax.axis_index("c")
    iota = _iota16()

    def load_start(c, buf, sem):
        for j in range(3):
            pltpu.async_copy(weT.at[:, pl.ds(c * W + j * 128, 128)],
                             buf.at[j], sem)

    def load_wait(buf, sem):
        for j in range(3):
            pltpu.make_async_copy(weT.at[:, pl.ds(0, 128)], buf.at[j],
                                  sem).wait()

    def store_start(c, tb, sem):
        pltpu.async_copy(tb, tbl2.at[pl.ds(c * WP, WP)], sem)

    def store_wait(tb, sem):
        pltpu.make_async_copy(tb, tbl2.at[pl.ds(0, WP)], sem).wait()

    def transpose(buf, tb):
        for j in range(3):

            @plsc.parallel_loop(0, 64, unroll=8)
            def pbody(p, j=j):
                for lg in range(4):
                    rvec = 16 * lg + iota
                    clo = jnp.full((16,), 2 * p, jnp.int32)
                    lo = plsc.load_gather(buf.at[j], [rvec, clo])
                    tb[64 * j + p, pl.ds(16 * lg, 16)] = lo
                    hi = plsc.load_gather(buf.at[j], [rvec, clo + 1])
                    tb[64 * j + p, pl.ds(64 + 16 * lg, 16)] = hi

    def valid(k):
        return k * NW + wid < NCH

    load_start(wid, buf_a, lsem_a)

    def body(i, carry):
        k_a, k_b = 2 * i, 2 * i + 1
        c_a = k_a * NW + wid
        c_b = k_b * NW + wid

        @pl.when(valid(k_b))
        def _():
            load_start(c_b, buf_b, lsem_b)

        load_wait(buf_a, lsem_a)

        @pl.when(i > 0)
        def _():
            store_wait(tb_a, ssem_a)

        transpose(buf_a, tb_a)
        store_start(c_a, tb_a, ssem_a)

        @pl.when(k_a + 2 < ITER_A)
        def _():
            load_start(c_a + 2 * NW, buf_a, lsem_a)

        @pl.when(valid(k_b))
        def _():
            @pl.when(i > 0)
            def _():
                store_wait(tb_b, ssem_b)

            load_wait(buf_b, lsem_b)
            transpose(buf_b, tb_b)
            store_start(c_b, tb_b, ssem_b)

        return carry

    lax.fori_loop(0, ITER_A // 2, body, 0)

    store_wait(tb_a, ssem_a)

    @pl.when(valid(ITER_A - 1))
    def _():
        store_wait(tb_b, ssem_b)

    @pl.when(wid == 0)
    def _():
        pltpu.sync_copy(tailp, buf_a.at[0])
        for lg in range(4):
            rvec = 16 * lg + iota

            def tbody(p, _, lg=lg, rvec=rvec):
                pvec = jnp.full((16,), 2 * p, jnp.int32)
                lo = plsc.load_gather(buf_a.at[0], [pvec, rvec])
                tb_a[p, pl.ds(16 * lg, 16)] = lo
                hi = plsc.load_gather(buf_a.at[0], [pvec + 1, rvec])
                tb_a[p, pl.ds(64 + 16 * lg, 16)] = hi
                return _

            lax.fori_loop(0, 32, tbody, 0)
        pltpu.sync_copy(tb_a.at[pl.ds(0, 32)], tbl2.at[pl.ds(QTAIL, 32)])


@functools.partial(
    pl.kernel,
    mesh=_mesh,
    out_type=jax.ShapeDtypeStruct((NT, D, NB), jnp.float32),
    scratch_types=[
        pltpu.VMEM((BLK,), jnp.int32),
        pltpu.VMEM((BLK,), jnp.int32),
        pltpu.VMEM((BLK,), jnp.int32),
        pltpu.VMEM((BLK,), jnp.int32),
        pltpu.VMEM((BLK, DP), jnp.float32),
        pltpu.VMEM((BLK, DP), jnp.float32),
        pltpu.VMEM((D, BLK), jnp.float32),
        pltpu.VMEM((D, BLK), jnp.float32),
        pltpu.SemaphoreType.DMA,
        pltpu.SemaphoreType.DMA,
        pltpu.SemaphoreType.DMA,
        pltpu.SemaphoreType.DMA,
    ],
    compiler_params=_params,
)
def _gather_kernel(idxT, tbl2, outT, iv_a, iv_b, qv_a, qv_b,
                   buf_a, buf_b, tb_a, tb_b,
                   gsem_a, gsem_b, ssem_a, ssem_b):
    wid = lax.axis_index("s") * NC + lax.axis_index("c")
    b0 = wid * BLK
    iota = _iota16()

    def fetch_start(u, iv, qv, buf, sem):
        pltpu.sync_copy(idxT.at[u, pl.ds(b0, BLK)], iv)
        for g in range(8):
            qv[pl.ds(16 * g, 16)] = iv[pl.ds(16 * g, 16)] >> 1
        pltpu.async_copy(tbl2.at[qv], buf, sem)

    def fetch_wait(buf, sem):
        pltpu.make_async_copy(tbl2.at[qv_a], buf, sem).wait()

    def store_start(u, tb, sem):
        pltpu.async_copy(tb, outT.at[u, :, pl.ds(b0, BLK)], sem)

    def store_wait(tb, sem):
        pltpu.make_async_copy(tb, outT.at[0, :, pl.ds(b0, BLK)], sem).wait()

    def transpose(buf, iv, tb):
        for jg in range(8):
            jvec = 16 * jg + iota
            parv = (iv[pl.ds(16 * jg, 16)] & 1) * 64

            @plsc.parallel_loop(0, D, unroll=8)
            def body(d, jg=jg, jvec=jvec, parv=parv):
                v = plsc.load_gather(buf, [jvec, parv + d])
                tb[d, pl.ds(16 * jg, 16)] = v

    fetch_start(0, iv_a, qv_a, buf_a, gsem_a)

    def body(i, carry):
        u_a, u_b = 2 * i, 2 * i + 1
        fetch_start(u_b, iv_b, qv_b, buf_b, gsem_b)
        fetch_wait(buf_a, gsem_a)

        @pl.when(i > 0)
        def _():
            store_wait(tb_a, ssem_a)

        transpose(buf_a, iv_a, tb_a)
        store_start(u_a, tb_a, ssem_a)

        @pl.when(u_a + 2 < NUNIT)
        def _():
            fetch_start(u_a + 2, iv_a, qv_a, buf_a, gsem_a)

        fetch_wait(buf_b, gsem_b)

        @pl.when(i > 0)
        def _():
            store_wait(tb_b, ssem_b)

        transpose(buf_b, iv_b, tb_b)
        store_start(u_b, tb_b, ssem_b)
        return carry

    lax.fori_loop(0, NUNIT // 2, body, 0)

    store_wait(tb_a, ssem_a)
    store_wait(tb_b, ssem_b)


def kernel(input, word_embedding):
    weT = word_embedding.T
    tailp = jnp.pad(word_embedding[VMAIN:], ((0, 0), (0, DP - D)))
    tbl2 = _pack_kernel(weT, tailp)
    idxT = input.astype(jnp.int32).T
    outT = _gather_kernel(idxT, tbl2)
    return outT.transpose(2, 0, 1)


# speedup vs baseline: 1.8870x; 1.0373x over previous
import functools

import jax
import jax.numpy as jnp
from jax import lax
from jax.experimental import pallas as pl
from jax.experimental.pallas import tpu as pltpu
from jax.experimental.pallas import tpu_sc as plsc

D = 64
DP = 128
NT = 200
NB = 4096
V = 1000000
VMAIN = 999936
NPAIR = V // 2
QTAIL = VMAIN // 2

_info = plsc.get_sparse_core_info()
NC, NS = _info.num_cores, _info.num_subcores
NW = NC * NS

W = 384
WP = W // 2
NCH = VMAIN // W
ITER_A = 82

BLK = 128
NUNIT = NT

_mesh = plsc.VectorSubcoreMesh(core_axis_name="c", subcore_axis_name="s")
_params = pltpu.CompilerParams(use_tc_tiling_on_sc=True,
                               needs_layout_passes=False,
                               disable_bounds_checks=True)


def _iota16():
    return lax.iota(jnp.int32, 16)


@functools.partial(
    pl.kernel,
    mesh=_mesh,
    out_type=jax.ShapeDtypeStruct((NPAIR, DP), jnp.float32),
    scratch_types=[
        pltpu.VMEM((3, D, 128), jnp.float32),
        pltpu.VMEM((3, D, 128), jnp.float32),
        pltpu.VMEM((WP, DP), jnp.float32),
        pltpu.VMEM((WP, DP), jnp.float32),
        pltpu.SemaphoreType.DMA,
        pltpu.SemaphoreType.DMA,
        pltpu.SemaphoreType.DMA,
        pltpu.SemaphoreType.DMA,
    ],
    compiler_params=_params,
)
def _pack_kernel(weT, tailp, tbl2, buf_a, buf_b, tb_a, tb_b,
                 lsem_a, lsem_b, ssem_a, ssem_b):
    wid = lax.axis_index("s") * NC + lax.axis_index("c")
    iota = _iota16()

    def load_start(c, buf, sem):
        for j in range(3):
            pltpu.async_copy(weT.at[:, pl.ds(c * W + j * 128, 128)],
                             buf.at[j], sem)

    def load_wait(buf, sem):
        for j in range(3):
            pltpu.make_async_copy(weT.at[:, pl.ds(0, 128)], buf.at[j],
                                  sem).wait()

    def store_start(c, tb, sem):
        pltpu.async_copy(tb, tbl2.at[pl.ds(c * WP, WP)], sem)

    def store_wait(tb, sem):
        pltpu.make_async_copy(tb, tbl2.at[pl.ds(0, WP)], sem).wait()

    def transpose(buf, tb):
        for j in range(3):

            @plsc.parallel_loop(0, 64, unroll=16)
            def pbody(p, j=j):
                for lg in range(4):
                    rvec = 16 * lg + iota
                    clo = jnp.full((16,), 2 * p, jnp.int32)
                    lo = plsc.load_gather(buf.at[j], [rvec, clo])
                    tb[64 * j + p, pl.ds(16 * lg, 16)] = lo
                    hi = plsc.load_gather(buf.at[j], [rvec, clo + 1])
                    tb[64 * j + p, pl.ds(64 + 16 * lg, 16)] = hi

    def valid(k):
        return k * NW + wid < NCH

    load_start(wid, buf_a, lsem_a)

    def body(i, carry):
        k_a, k_b = 2 * i, 2 * i + 1
        c_a = k_a * NW + wid
        c_b = k_b * NW + wid

        @pl.when(valid(k_b))
        def _():
            load_start(c_b, buf_b, lsem_b)

        load_wait(buf_a, lsem_a)

        @pl.when(i > 0)
        def _():
            store_wait(tb_a, ssem_a)

        transpose(buf_a, tb_a)
        store_start(c_a, tb_a, ssem_a)

        @pl.when(k_a + 2 < ITER_A)
        def _():
            load_start(c_a + 2 * NW, buf_a, lsem_a)

        @pl.when(valid(k_b))
        def _():
            @pl.when(i > 0)
            def _():
                store_wait(tb_b, ssem_b)

            load_wait(buf_b, lsem_b)
            transpose(buf_b, tb_b)
            store_start(c_b, tb_b, ssem_b)

        return carry

    lax.fori_loop(0, ITER_A // 2, body, 0)

    store_wait(tb_a, ssem_a)

    @pl.when(valid(ITER_A - 1))
    def _():
        store_wait(tb_b, ssem_b)

    @pl.when(wid == 0)
    def _():
        pltpu.sync_copy(tailp, buf_a.at[0])
        for lg in range(4):
            rvec = 16 * lg + iota

            def tbody(p, _, lg=lg, rvec=rvec):
                pvec = jnp.full((16,), 2 * p, jnp.int32)
                lo = plsc.load_gather(buf_a.at[0], [pvec, rvec])
                tb_a[p, pl.ds(16 * lg, 16)] = lo
                hi = plsc.load_gather(buf_a.at[0], [pvec + 1, rvec])
                tb_a[p, pl.ds(64 + 16 * lg, 16)] = hi
                return _

            lax.fori_loop(0, 32, tbody, 0)
        pltpu.sync_copy(tb_a.at[pl.ds(0, 32)], tbl2.at[pl.ds(QTAIL, 32)])


@functools.partial(
    pl.kernel,
    mesh=_mesh,
    out_type=jax.ShapeDtypeStruct((NT, D, NB), jnp.float32),
    scratch_types=[
        pltpu.VMEM((NUNIT, BLK), jnp.int32),
        pltpu.VMEM((BLK,), jnp.int32),
        pltpu.VMEM((BLK,), jnp.int32),
        pltpu.VMEM((BLK, DP), jnp.float32),
        pltpu.VMEM((BLK, DP), jnp.float32),
        pltpu.VMEM((D, BLK), jnp.float32),
        pltpu.VMEM((D, BLK), jnp.float32),
        pltpu.SemaphoreType.DMA,
        pltpu.SemaphoreType.DMA,
        pltpu.SemaphoreType.DMA,
        pltpu.SemaphoreType.DMA,
    ],
    compiler_params=_params,
)
def _gather_kernel(idxT, tbl2, outT, iv_all, qv_a, qv_b,
                   buf_a, buf_b, tb_a, tb_b,
                   gsem_a, gsem_b, ssem_a, ssem_b):
    wid = lax.axis_index("s") * NC + lax.axis_index("c")
    b0 = wid * BLK
    iota = _iota16()

    pltpu.sync_copy(idxT.at[:, pl.ds(b0, BLK)], iv_all)

    def fetch_start(u, qv, buf, sem):
        for g in range(8):
            qv[pl.ds(16 * g, 16)] = iv_all[u, pl.ds(16 * g, 16)] >> 1
        pltpu.async_copy(tbl2.at[qv], buf, sem)

    def fetch_wait(buf, sem):
        pltpu.make_async_copy(tbl2.at[qv_a], buf, sem).wait()

    def store_start(u, tb, sem):
        pltpu.async_copy(tb, outT.at[u, :, pl.ds(b0, BLK)], sem)

    def store_wait(tb, sem):
        pltpu.make_async_copy(tb, outT.at[0, :, pl.ds(b0, BLK)], sem).wait()

    def transpose(buf, u, tb):
        for jg in range(8):
            jvec = 16 * jg + iota
            parv = (iv_all[u, pl.ds(16 * jg, 16)] & 1) * 64

            @plsc.parallel_loop(0, D, unroll=16)
            def body(d, jg=jg, jvec=jvec, parv=parv):
                v = plsc.load_gather(buf, [jvec, parv + d])
                tb[d, pl.ds(16 * jg, 16)] = v

    fetch_start(0, qv_a, buf_a, gsem_a)

    def body(i, carry):
        u_a, u_b = 2 * i, 2 * i + 1
        fetch_start(u_b, qv_b, buf_b, gsem_b)
        fetch_wait(buf_a, gsem_a)

        @pl.when(i > 0)
        def _():
            store_wait(tb_a, ssem_a)

        transpose(buf_a, u_a, tb_a)
        store_start(u_a, tb_a, ssem_a)

        @pl.when(u_a + 2 < NUNIT)
        def _():
            fetch_start(u_a + 2, qv_a, buf_a, gsem_a)

        fetch_wait(buf_b, gsem_b)

        @pl.when(i > 0)
        def _():
            store_wait(tb_b, ssem_b)

        transpose(buf_b, u_b, tb_b)
        store_start(u_b, tb_b, ssem_b)
        return carry

    lax.fori_loop(0, NUNIT // 2, body, 0)

    store_wait(tb_a, ssem_a)
    store_wait(tb_b, ssem_b)


def kernel(input, word_embedding):
    weT = word_embedding.T
    tailp = jnp.pad(word_embedding[VMAIN:], ((0, 0), (0, DP - D)))
    tbl2 = _pack_kernel(weT, tailp)
    idxT = input.astype(jnp.int32).T
    outT = _gather_kernel(idxT, tbl2)
    return outT.transpose(2, 0, 1)


# speedup vs baseline: 2.8594x; 1.5153x over previous
import functools

import jax
import jax.numpy as jnp
from jax import lax
from jax.experimental import pallas as pl
from jax.experimental.pallas import tpu as pltpu
from jax.experimental.pallas import tpu_sc as plsc

D = 64
DP = 128
B_TOTAL = 4096 * 200

_info = plsc.get_sparse_core_info()
NC, NS = _info.num_cores, _info.num_subcores
NW = NC * NS
PER_W = B_TOTAL // NW
CHUNK = 320
NCHUNK = PER_W // CHUNK

_mesh = plsc.VectorSubcoreMesh(core_axis_name="c", subcore_axis_name="s")


@functools.partial(
    pl.kernel,
    mesh=_mesh,
    out_type=jax.ShapeDtypeStruct((B_TOTAL, DP), jnp.float32),
    scratch_types=[
        pltpu.VMEM((PER_W,), jnp.int32),
        pltpu.VMEM((CHUNK, DP), jnp.float32),
        pltpu.VMEM((CHUNK, DP), jnp.float32),
        pltpu.SemaphoreType.DMA,
        pltpu.SemaphoreType.DMA,
    ],
    compiler_params=pltpu.CompilerParams(use_tc_tiling_on_sc=True),
)
def _gather_kernel(idx_hbm, table_hbm, out_hbm, idx_v, buf_a, buf_b, sem_a, sem_b):
    wid = lax.axis_index("s") * NC + lax.axis_index("c")
    base = wid * PER_W

    pltpu.sync_copy(idx_hbm.at[pl.ds(base, PER_W)], idx_v)

    def gather_start(c, buf, sem):
        pltpu.async_copy(
            table_hbm.at[idx_v.at[pl.ds(c * CHUNK, CHUNK)]], buf, sem)

    def gather_wait(buf, sem):
        pltpu.make_async_copy(
            table_hbm.at[idx_v.at[pl.ds(0, CHUNK)]], buf, sem).wait()

    def store(c, buf):
        pltpu.sync_copy(buf, out_hbm.at[pl.ds(base + c * CHUNK, CHUNK)])

    gather_start(0, buf_a, sem_a)

    def body(g, carry):
        gather_start(g + 1, buf_b, sem_b)
        gather_wait(buf_a, sem_a)
        store(g, buf_a)

        @pl.when(g + 2 < NCHUNK)
        def _():
            gather_start(g + 2, buf_a, sem_a)

        gather_wait(buf_b, sem_b)
        store(g + 1, buf_b)
        return carry

    lax.fori_loop(0, NCHUNK // 2, lambda i, c: body(2 * i, c), 0)


def kernel(input, word_embedding):
    idx = input.reshape(-1).astype(jnp.int32)
    table_p = jnp.pad(word_embedding, ((0, 0), (0, DP - D)))
    out = _gather_kernel(idx, table_p)
    return out[:, :D].reshape(input.shape + (D,))
